# baseline (speedup 1.0000x reference)
import jax
import jax.numpy as jnp
from jax import lax
from jax.experimental import pallas as pl
from jax.experimental.pallas import tpu as pltpu
from jax.experimental.pallas import tpu_sc as plsc

R = 16384
S = 64
NC, NS = 2, 16
NW = NC * NS
RPW = R // NW


def _body(u_hbm, out_hbm):
    wid = lax.axis_index("s") * NC + lax.axis_index("c")
    base = wid * RPW
    pltpu.sync_copy(u_hbm.at[pl.ds(base, RPW)], out_hbm.at[pl.ds(base, RPW)])


_probe = pl.kernel(
    _body,
    out_type=jax.ShapeDtypeStruct((R, S), jnp.float32),
    mesh=plsc.VectorSubcoreMesh(
        core_axis_name="c", subcore_axis_name="s", num_cores=NC, num_subcores=NS
    ),
    scratch_types=[],
    compiler_params=pltpu.CompilerParams(
        needs_layout_passes=False, use_tc_tiling_on_sc=True
    ),
)


def kernel(deltas, density, bins, u):
    return _probe(u)


# speedup vs baseline: 1.7678x; 1.7678x over previous
import jax
import jax.numpy as jnp
from jax import lax
from jax.experimental import pallas as pl
from jax.experimental.pallas import tpu as pltpu
from jax.experimental.pallas import tpu_sc as plsc

R = 16384
C = 64
S = 64
EPS = 1e-5

NC, NS = 2, 16
NW = NC * NS
RPW = R // NW
CH = 256
NCH = RPW // CH

OFF_DEL = 0
OFF_DEN = OFF_DEL + R * C
OFF_BIN = OFF_DEN + R * C
OFF_U = OFF_BIN + R * (C + 1)

_DN = lax.GatherDimensionNumbers(
    offset_dims=(), collapsed_slice_dims=(0,), start_index_map=(0,)
)


def _dg(x, i):
    return lax.gather(x, i[:, None], _DN, slice_sizes=(1,),
                      mode=lax.GatherScatterMode.PROMISE_IN_BOUNDS)


def _body(in_hbm, out_hbm, del_v, den_v, bins_v, u_v, cdf_v, out_v):
    wid = lax.axis_index("s") * NC + lax.axis_index("c")
    lanes = lax.broadcasted_iota(jnp.int32, (16,), 0)
    flanes = lanes.astype(jnp.float32)
    l15 = jnp.full((16,), 15, jnp.int32)

    for k in range(NCH):
        base = wid * RPW + k * CH
        pltpu.sync_copy(in_hbm.at[pl.ds(OFF_DEL + base * C, CH * C)], del_v)
        pltpu.sync_copy(in_hbm.at[pl.ds(OFF_DEN + base * C, CH * C)], den_v)
        pltpu.sync_copy(
            in_hbm.at[pl.ds(OFF_BIN + base * (C + 1), CH * (C + 1))], bins_v)
        pltpu.sync_copy(in_hbm.at[pl.ds(OFF_U + base * S, CH * S)], u_v)

        @plsc.parallel_loop(0, CH, 1, unroll=2)
        def ray_body(r):
            ro = r * C
            bo = r * (C + 1)
            css = []
            for c in range(4):
                sl = pl.ds(ro + c * 16, 16)
                css.append(plsc.cumsum(del_v[sl] * den_v[sl]))
            carry = jnp.zeros((16,), jnp.float32)
            Es = []
            for c in range(4):
                T = css[c] + carry
                carry = _dg(T, l15)
                Es.append(jnp.exp(-T))
            ws = 1.0 - _dg(Es[3], l15)
            pad = jnp.maximum(EPS - ws, 0.0)
            p64 = pad * (1.0 / 64.0)
            inv = 1.0 / (ws + pad)
            vals = []
            for c in range(4):
                j1 = flanes + jnp.float32(c * 16 + 1)
                v = jnp.minimum((1.0 - Es[c] + j1 * p64) * inv, 1.0)
                vals.append(v)
                cdf_v[pl.ds(ro + c * 16, 16)] = v
            t0 = _dg(vals[0], l15)
            t1 = _dg(vals[1], l15)
            t2 = _dg(vals[2], l15)
            for sb in range(4):
                u = u_v[pl.ds(ro + sb * 16, 16)]
                cnt = (jnp.where(t0 <= u, 16, 0)
                       + jnp.where(t1 <= u, 16, 0)
                       + jnp.where(t2 <= u, 16, 0))
                for step in (8, 4, 2, 1, 1):
                    v = plsc.load_gather(cdf_v, [cnt + (ro + step - 1)])
                    cnt = jnp.where(v <= u, cnt + step, cnt)
                g0 = plsc.load_gather(cdf_v, [jnp.maximum(cnt - 1, 0) + ro])
                g0 = jnp.where(cnt == 0, 0.0, g0)
                g1 = plsc.load_gather(cdf_v, [jnp.minimum(cnt, 63) + ro])
                b0 = plsc.load_gather(bins_v, [cnt + bo])
                b1 = plsc.load_gather(bins_v, [jnp.minimum(cnt + 1, 64) + bo])
                denom = g1 - g0
                denom = jnp.where(denom < EPS, 1.0, denom)
                t = (u - g0) / denom
                out_v[pl.ds(ro + sb * 16, 16)] = b0 + t * (b1 - b0)

        pltpu.sync_copy(out_v, out_hbm.at[pl.ds(base * S, CH * S)])


_sampler = pl.kernel(
    _body,
    out_type=jax.ShapeDtypeStruct((R * S,), jnp.float32),
    mesh=plsc.VectorSubcoreMesh(
        core_axis_name="c", subcore_axis_name="s", num_cores=NC, num_subcores=NS
    ),
    scratch_types=[
        pltpu.VMEM((CH * C,), jnp.float32),
        pltpu.VMEM((CH * C,), jnp.float32),
        pltpu.VMEM((CH * (C + 1),), jnp.float32),
        pltpu.VMEM((CH * S,), jnp.float32),
        pltpu.VMEM((CH * C,), jnp.float32),
        pltpu.VMEM((CH * S,), jnp.float32),
    ],
    compiler_params=pltpu.CompilerParams(needs_layout_passes=False),
)


def kernel(deltas, density, bins, u):
    flat = jnp.concatenate([
        deltas.reshape(-1),
        density.reshape(-1),
        bins.reshape(-1),
        u.reshape(-1),
    ])
    return _sampler(flat).reshape(R, S)


# speedup vs baseline: 2.5806x; 1.4598x over previous
import jax
import jax.numpy as jnp
from jax import lax
from jax.experimental import pallas as pl
from jax.experimental.pallas import tpu as pltpu
from jax.experimental.pallas import tpu_sc as plsc

R = 16384
C = 64
S = 64
EPS = 1e-5

NC, NS = 2, 16
NW = NC * NS
RPW = R // NW
CH = 128
NCH = RPW // CH

_DN = lax.GatherDimensionNumbers(
    offset_dims=(), collapsed_slice_dims=(0,), start_index_map=(0,)
)


def _dg(x, i):
    return lax.gather(x, i[:, None], _DN, slice_sizes=(1,),
                      mode=lax.GatherScatterMode.PROMISE_IN_BOUNDS)


def _body(del_hbm, den_hbm, bins_hbm, u_hbm, out_hbm,
          del_v, den_v, bins_v, u_v, cdf_v, out_v):
    wid = lax.axis_index("s") * NC + lax.axis_index("c")
    lanes = lax.broadcasted_iota(jnp.int32, (16,), 0)
    flanes = lanes.astype(jnp.float32)
    l15 = jnp.full((16,), 15, jnp.int32)

    for k in range(NCH):
        base = wid * RPW + k * CH
        pltpu.sync_copy(del_hbm.at[pl.ds(base, CH)], del_v)
        pltpu.sync_copy(den_hbm.at[pl.ds(base, CH)], den_v)
        pltpu.sync_copy(bins_hbm.at[pl.ds(base, CH)], bins_v)
        pltpu.sync_copy(u_hbm.at[pl.ds(base, CH)], u_v)

        @plsc.parallel_loop(0, CH, 1, unroll=2 if NCH <= 2 else 1)
        def ray_body(r):
            ro = r * C
            rv = jnp.full((16,), r, jnp.int32)
            css = []
            for c in range(4):
                sl = pl.ds(c * 16, 16)
                css.append(plsc.cumsum(del_v[r, sl] * den_v[r, sl]))
            carry = jnp.zeros((16,), jnp.float32)
            Es = []
            for c in range(4):
                T = css[c] + carry
                carry = _dg(T, l15)
                Es.append(jnp.exp(-T))
            ws = 1.0 - _dg(Es[3], l15)
            pad = jnp.maximum(EPS - ws, 0.0)
            p64 = pad * (1.0 / 64.0)
            inv = 1.0 / (ws + pad)
            vals = []
            for c in range(4):
                j1 = flanes + jnp.float32(c * 16 + 1)
                v = jnp.minimum((1.0 - Es[c] + j1 * p64) * inv, 1.0)
                vals.append(v)
                cdf_v[pl.ds(ro + c * 16, 16)] = v
            t0 = _dg(vals[0], l15)
            t1 = _dg(vals[1], l15)
            t2 = _dg(vals[2], l15)
            for sb in range(4):
                u = u_v[r, pl.ds(sb * 16, 16)]
                cnt = (jnp.where(t0 <= u, 16, 0)
                       + jnp.where(t1 <= u, 16, 0)
                       + jnp.where(t2 <= u, 16, 0))
                for step in (8, 4, 2, 1, 1):
                    v = plsc.load_gather(cdf_v, [cnt + (ro + step - 1)])
                    cnt = jnp.where(v <= u, cnt + step, cnt)
                g0 = plsc.load_gather(cdf_v, [jnp.maximum(cnt - 1, 0) + ro])
                g0 = jnp.where(cnt == 0, 0.0, g0)
                g1 = plsc.load_gather(cdf_v, [jnp.minimum(cnt, 63) + ro])
                b0 = plsc.load_gather(bins_v, [rv, cnt])
                b1 = plsc.load_gather(bins_v, [rv, jnp.minimum(cnt + 1, 64)])
                denom = g1 - g0
                denom = jnp.where(denom < EPS, 1.0, denom)
                t = (u - g0) / denom
                out_v[r, pl.ds(sb * 16, 16)] = b0 + t * (b1 - b0)

        pltpu.sync_copy(out_v, out_hbm.at[pl.ds(base, CH)])


_sampler = pl.kernel(
    _body,
    out_type=jax.ShapeDtypeStruct((R, S), jnp.float32),
    mesh=plsc.VectorSubcoreMesh(
        core_axis_name="c", subcore_axis_name="s", num_cores=NC, num_subcores=NS
    ),
    scratch_types=[
        pltpu.VMEM((CH, C), jnp.float32),
        pltpu.VMEM((CH, C), jnp.float32),
        pltpu.VMEM((CH, C + 1), jnp.float32),
        pltpu.VMEM((CH, S), jnp.float32),
        pltpu.VMEM((CH * C,), jnp.float32),
        pltpu.VMEM((CH, S), jnp.float32),
    ],
    compiler_params=pltpu.CompilerParams(
        needs_layout_passes=False, use_tc_tiling_on_sc=True
    ),
)


def kernel(deltas, density, bins, u):
    return _sampler(deltas, density[..., 0], bins, u)
